# baseline (speedup 1.0000x reference)
import functools

import jax
import jax.numpy as jnp
from jax import lax
from jax.experimental import pallas as pl
from jax.experimental.pallas import tpu as pltpu
from jax.experimental.pallas import tpu_sc as plsc

PAST = 24
FUTURE = 12
HEADS = 4
D1 = 26


def _make_gat_edge_sc(N, E, W, CH, OUTC):
    NT = 16
    CHUNK = 64
    N2 = N // 2
    rpt = N2 // NT
    ept = E // NT
    nchunks = ept // CHUNK
    assert rpt * NT == N2 and nchunks * CHUNK == ept and nchunks % 2 == 0
    mesh = plsc.VectorSubcoreMesh(core_axis_name="c", subcore_axis_name="s")

    @functools.partial(
        pl.kernel,
        mesh=mesh,
        out_type=jax.ShapeDtypeStruct((N, W), jnp.float32),
        compiler_params=pltpu.CompilerParams(
            needs_layout_passes=False, use_tc_tiling_on_sc=False),
        scratch_types=[
            pltpu.VMEM_SHARED((N2, W), jnp.float32),
            pltpu.VMEM((2 * CHUNK,), jnp.int32),
            pltpu.VMEM((2 * CHUNK,), jnp.int32),
            pltpu.VMEM((CHUNK,), jnp.int32),
            pltpu.VMEM((CHUNK,), jnp.int32),
            pltpu.VMEM((CHUNK,), jnp.int32),
            pltpu.VMEM((CHUNK,), jnp.int32),
            pltpu.VMEM((CHUNK,), jnp.int32),
            pltpu.VMEM((CHUNK,), jnp.int32),
            pltpu.VMEM((CHUNK,), jnp.float32),
            pltpu.VMEM((CHUNK,), jnp.float32),
            pltpu.VMEM((CHUNK, W), jnp.float32),
            pltpu.VMEM((CHUNK, W), jnp.float32),
            pltpu.VMEM((CHUNK, 16), jnp.float32),
            pltpu.VMEM((CHUNK, 16), jnp.float32),
            pltpu.VMEM((CHUNK, W), jnp.float32),
            pltpu.VMEM((CHUNK, W), jnp.float32),
            pltpu.SemaphoreType.DMA, pltpu.SemaphoreType.DMA,
            pltpu.SemaphoreType.DMA, pltpu.SemaphoreType.DMA,
            pltpu.SemaphoreType.DMA, pltpu.SemaphoreType.DMA,
        ],
    )
    def gat_edge(ht, at, si, epk, acc,
                 slab, idxb0, idxb1, sidx0, sidx1, didx0, didx1, gdix0, gdix1,
                 ownb0, ownb1, srows0, srows1, drows0, drows1, obuf0, obuf1,
                 semi0, semi1, semg0, semg1, semsc0, semsc1):
        gdix = (gdix0, gdix1)
        idxb = (idxb0, idxb1)
        sidx = (sidx0, sidx1)
        didx = (didx0, didx1)
        ownb = (ownb0, ownb1)
        srows = (srows0, srows1)
        drows = (drows0, drows1)
        obuf = (obuf0, obuf1)
        semi = (semi0, semi1)
        semg = (semg0, semg1)
        semsc = (semsc0, semsc1)
        c = lax.axis_index("c")
        s = lax.axis_index("s")
        r0 = s * rpt
        cN2 = c * N2
        pltpu.sync_copy(si.at[pl.ds(cN2 + r0, rpt)], slab.at[pl.ds(r0, rpt)])

        for b in range(2):
            def zrow(r, carry, _ob=obuf[b]):
                for vb in range(W // 16):
                    _ob[r, pl.ds(vb * 16, 16)] = jnp.zeros((16,), jnp.float32)
                return carry
            lax.fori_loop(0, CHUNK, zrow, 0)
        plsc.subcore_barrier()

        g0 = s * nchunks

        def start_idx(i, b):
            pltpu.async_copy(
                epk.at[pl.ds((g0 + i) * (2 * CHUNK), 2 * CHUNK)], idxb[b], semi[b])

        def wait_idx(b):
            pltpu.make_async_copy(
                epk.at[pl.ds(0, 2 * CHUNK)], idxb[b], semi[b]).wait()

        def extract(b):
            for g in range(CHUNK // 16):
                sidx[b][pl.ds(g * 16, 16)] = idxb[b][pl.ds(g * 16, 16)]
            for g in range(CHUNK // 16):
                dg = idxb[b][pl.ds(CHUNK + g * 16, 16)]
                gdix[b][pl.ds(g * 16, 16)] = dg
                dv = dg - cN2
                own = (dv >= 0) & (dv < N2)
                ownb[b][pl.ds(g * 16, 16)] = jnp.where(own, 1.0, 0.0)
                didx[b][pl.ds(g * 16, 16)] = jnp.clip(dv, 0, N2 - 1)

        def start_gather(b):
            pltpu.async_copy(ht.at[sidx[b]], srows[b], semg[b])
            pltpu.async_copy(at.at[gdix[b]], drows[b], semg[b])

        def wait_gather(b):
            pltpu.make_async_copy(ht.at[sidx[b]], srows[b], semg[b]).wait()
            pltpu.make_async_copy(at.at[gdix[b]], drows[b], semg[b]).wait()

        def wait_scat(b):
            pltpu.make_async_copy(obuf[b], slab.at[didx[b]], semsc[b]).wait()

        def compute(b):
            def group(g, cc):
                ids = lax.iota(jnp.int32, 16) + g * 16
                ownv = ownb[b][pl.ds(g * 16, 16)]
                ps = []
                for h in range(HEADS):
                    colh = jnp.zeros((16,), jnp.int32) + (CH + h)
                    asrc = plsc.load_gather(srows[b], [ids, colh])
                    cold = jnp.zeros((16,), jnp.int32) + h
                    adst = plsc.load_gather(drows[b], [ids, cold])
                    e = asrc + adst
                    e = jnp.where(e >= 0.0, e, 0.2 * e)
                    p = jnp.exp(e) * ownv
                    ps.append(p)
                    plsc.store_scatter(obuf[b], [ids, colh], p)
                for j in range(CH):
                    colj = jnp.zeros((16,), jnp.int32) + j
                    v = plsc.load_gather(srows[b], [ids, colj])
                    plsc.store_scatter(obuf[b], [ids, colj], v * ps[j // OUTC])
                return cc
            lax.fori_loop(0, CHUNK // 16, group, 0)

        start_idx(0, 0)
        wait_idx(0)
        extract(0)
        start_gather(0)
        start_idx(1, 1)
        start_idx(2, 0)

        def body(i2, carry):
            for b in range(2):
                i = i2 * 2 + b
                nb = 1 - b

                @pl.when(i + 1 < nchunks)
                def _prep():
                    wait_idx(nb)

                    @pl.when(i >= 1)
                    def _():
                        wait_scat(nb)
                    extract(nb)
                    start_gather(nb)

                    @pl.when(i + 3 < nchunks)
                    def _():
                        start_idx(i + 3, nb)

                wait_gather(b)
                compute(b)
                pltpu.async_copy(obuf[b], slab.at[didx[b]], semsc[b], add=True)
            return carry
        lax.fori_loop(0, nchunks // 2, body, 0)
        wait_scat(0)
        wait_scat(1)

        plsc.subcore_barrier()
        pltpu.sync_copy(slab.at[pl.ds(r0, rpt)], acc.at[pl.ds(cN2 + r0, rpt)])

    return gat_edge


def _mm_kernel(a_ref, b_ref, o_ref):
    o_ref[...] = jnp.dot(a_ref[...], b_ref[...], preferred_element_type=jnp.float32)


def _pallas_matmul(a, b, bm=512):
    M, K = a.shape
    _, N = b.shape
    assert M % bm == 0
    return pl.pallas_call(
        _mm_kernel,
        grid=(M // bm,),
        in_specs=[
            pl.BlockSpec((bm, K), lambda i: (i, 0)),
            pl.BlockSpec((K, N), lambda i: (0, 0)),
        ],
        out_specs=pl.BlockSpec((bm, N), lambda i: (i, 0)),
        out_shape=jax.ShapeDtypeStruct((M, N), jnp.float32),
    )(a, b)


def _leaky(x):
    return jnp.where(x >= 0, x, 0.2 * x)


def _gat_sc(x, edge_index, W, a_s, a_d, b, outc):
    N = x.shape[0]
    E = edge_index.shape[1]
    ch = HEADS * outc
    chh = ch // 2
    Wt = W.reshape(W.shape[0], HEADS, outc)
    As = jnp.einsum('khc,hc->kh', Wt, a_s)
    Ad = jnp.einsum('khc,hc->kh', Wt, a_d)
    hh = _pallas_matmul(x, jnp.concatenate([W, As, Ad], axis=1))
    h = hh[:, :ch]
    als = hh[:, ch:ch + HEADS]
    ald = hh[:, ch + HEADS:ch + 2 * HEADS]
    self_ex = jnp.exp(_leaky(als + ald))

    Wrow = -(-(ch + HEADS) // 16) * 16
    zpad = jnp.zeros((N, Wrow - ch - HEADS), jnp.float32)
    ht = jnp.concatenate([h, als, zpad], axis=1)
    at = jnp.concatenate([ald, jnp.zeros((N, 12), jnp.float32)], axis=1)
    se_rep = jnp.repeat(self_ex, outc, axis=1)
    si = jnp.concatenate([h * se_rep, self_ex, zpad], axis=1)

    epk = jnp.concatenate([
        edge_index[0].astype(jnp.int32).reshape(-1, 64),
        edge_index[1].astype(jnp.int32).reshape(-1, 64),
    ], axis=1).reshape(-1)
    acc = _make_gat_edge_sc(N, E, Wrow, ch, outc)(ht, at, si, epk)

    num = acc[:, :ch]
    den_rep = jnp.repeat(acc[:, ch:ch + HEADS], outc, axis=1) + 1e-16
    out = (num / den_rep).reshape(N, HEADS, outc).mean(axis=1) + b
    return out


def kernel(x_sg1, x_sg2, edge_index_sg1, edge_index_sg2, emb0, emb1,
           W1, a1_src, a1_dst, b1, W2, a2_src, a2_dst, b2, Wq, A):
    B = x_sg1.shape[0] // PAST
    i0a = x_sg1[:, 0].astype(jnp.int32)
    i1a = x_sg1[:, 1].astype(jnp.int32)
    x = jnp.concatenate([emb0[i0a], emb1[i1a], x_sg1[:, -3:]], axis=-1)

    x1 = _gat_sc(x, edge_index_sg1, W1, a1_src, a1_dst, b1, D1)

    i0b = x_sg2[:, 0].astype(jnp.int32)
    i1b = x_sg2[:, 1].astype(jnp.int32)
    x2_26 = jnp.concatenate([emb0[i0b], emb1[i1b], x_sg2[:, 2:4]], axis=-1)

    g1 = _pallas_matmul(x1, Wq).reshape(B, PAST, -1)
    g2 = _pallas_matmul(x2_26, Wq).reshape(B, FUTURE, -1)
    n1 = (g1 * g1).sum(-1)
    n2 = (g2 * g2).sum(-1)
    cr = jnp.einsum('bfk,bpk->bfp', g2, g1)
    logits = -(n2[:, :, None] + n1[:, None, :] - 2.0 * cr)
    A_tmp = A[:PAST, PAST:].T
    logits = jnp.where(A_tmp[None] == 0, -jnp.inf, logits)
    m = logits.max(axis=-1, keepdims=True)
    ea = jnp.exp(logits - m)
    alpha = ea / ea.sum(axis=-1, keepdims=True)
    y_past = x_sg1[:, 4].reshape(B, PAST)
    tmp = jnp.einsum('bfp,bp->bf', alpha, y_past)

    x2b = jnp.concatenate([x2_26, tmp.reshape(-1, 1)], axis=-1)
    out = _gat_sc(x2b, edge_index_sg2, W2, a2_src, a2_dst, b2, 1)
    return out.reshape(-1, FUTURE)


# speedup vs baseline: 1.0055x; 1.0055x over previous
import functools

import jax
import jax.numpy as jnp
from jax import lax
from jax.experimental import pallas as pl
from jax.experimental.pallas import tpu as pltpu
from jax.experimental.pallas import tpu_sc as plsc

PAST = 24
FUTURE = 12
HEADS = 4
D1 = 26


def _make_gat_edge_sc(N, E, W, CH, OUTC):
    NT = 16
    CHUNK = 64
    N2 = N // 2
    rpt = N2 // NT
    ept = E // NT
    nchunks = ept // CHUNK
    assert rpt * NT == N2 and nchunks * CHUNK == ept and nchunks % 2 == 0
    mesh = plsc.VectorSubcoreMesh(core_axis_name="c", subcore_axis_name="s")

    @functools.partial(
        pl.kernel,
        mesh=mesh,
        out_type=jax.ShapeDtypeStruct((N, W), jnp.float32),
        compiler_params=pltpu.CompilerParams(
            needs_layout_passes=False, use_tc_tiling_on_sc=False),
        scratch_types=[
            pltpu.VMEM_SHARED((N2, W), jnp.float32),
            pltpu.VMEM((2 * CHUNK,), jnp.int32),
            pltpu.VMEM((2 * CHUNK,), jnp.int32),
            pltpu.VMEM((CHUNK,), jnp.int32),
            pltpu.VMEM((CHUNK,), jnp.int32),
            pltpu.VMEM((CHUNK,), jnp.int32),
            pltpu.VMEM((CHUNK,), jnp.int32),
            pltpu.VMEM((CHUNK,), jnp.int32),
            pltpu.VMEM((CHUNK,), jnp.int32),
            pltpu.VMEM((CHUNK,), jnp.float32),
            pltpu.VMEM((CHUNK,), jnp.float32),
            pltpu.VMEM((CHUNK, W), jnp.float32),
            pltpu.VMEM((CHUNK, W), jnp.float32),
            pltpu.VMEM((CHUNK, 16), jnp.float32),
            pltpu.VMEM((CHUNK, 16), jnp.float32),
            pltpu.VMEM((CHUNK, W), jnp.float32),
            pltpu.VMEM((CHUNK, W), jnp.float32),
            pltpu.SemaphoreType.DMA, pltpu.SemaphoreType.DMA,
            pltpu.SemaphoreType.DMA, pltpu.SemaphoreType.DMA,
            pltpu.SemaphoreType.DMA, pltpu.SemaphoreType.DMA,
        ],
    )
    def gat_edge(ht, at, si, epk, acc,
                 slab, idxb0, idxb1, sidx0, sidx1, didx0, didx1, gdix0, gdix1,
                 ownb0, ownb1, srows0, srows1, drows0, drows1, obuf0, obuf1,
                 semi0, semi1, semg0, semg1, semsc0, semsc1):
        gdix = (gdix0, gdix1)
        idxb = (idxb0, idxb1)
        sidx = (sidx0, sidx1)
        didx = (didx0, didx1)
        ownb = (ownb0, ownb1)
        srows = (srows0, srows1)
        drows = (drows0, drows1)
        obuf = (obuf0, obuf1)
        semi = (semi0, semi1)
        semg = (semg0, semg1)
        semsc = (semsc0, semsc1)
        c = lax.axis_index("c")
        s = lax.axis_index("s")
        r0 = s * rpt
        cN2 = c * N2
        pltpu.sync_copy(si.at[pl.ds(cN2 + r0, rpt)], slab.at[pl.ds(r0, rpt)])

        for b in range(2):
            def zrow(r, carry, _ob=obuf[b]):
                for vb in range(W // 16):
                    _ob[r, pl.ds(vb * 16, 16)] = jnp.zeros((16,), jnp.float32)
                return carry
            lax.fori_loop(0, CHUNK, zrow, 0)
        plsc.subcore_barrier()

        g0 = s * nchunks

        def start_idx(i, b):
            pltpu.async_copy(
                epk.at[pl.ds((g0 + i) * (2 * CHUNK), 2 * CHUNK)], idxb[b], semi[b])

        def wait_idx(b):
            pltpu.make_async_copy(
                epk.at[pl.ds(0, 2 * CHUNK)], idxb[b], semi[b]).wait()

        def extract(b):
            for g in range(CHUNK // 16):
                sidx[b][pl.ds(g * 16, 16)] = idxb[b][pl.ds(g * 16, 16)]
            for g in range(CHUNK // 16):
                dg = idxb[b][pl.ds(CHUNK + g * 16, 16)]
                gdix[b][pl.ds(g * 16, 16)] = dg
                dv = dg - cN2
                own = (dv >= 0) & (dv < N2)
                ownb[b][pl.ds(g * 16, 16)] = jnp.where(own, 1.0, 0.0)
                didx[b][pl.ds(g * 16, 16)] = jnp.clip(dv, 0, N2 - 1)

        def start_gather(b):
            pltpu.async_copy(ht.at[sidx[b]], srows[b], semg[b])
            pltpu.async_copy(at.at[gdix[b]], drows[b], semg[b])

        def wait_gather(b):
            pltpu.make_async_copy(ht.at[sidx[b]], srows[b], semg[b]).wait()
            pltpu.make_async_copy(at.at[gdix[b]], drows[b], semg[b]).wait()

        def wait_scat(b):
            pltpu.make_async_copy(obuf[b], slab.at[didx[b]], semsc[b]).wait()

        def compute(b):
            def group(g, cc):
                ids = lax.iota(jnp.int32, 16) + g * 16
                ownv = ownb[b][pl.ds(g * 16, 16)]
                ps = []
                for h in range(HEADS):
                    colh = jnp.zeros((16,), jnp.int32) + (CH + h)
                    asrc = plsc.load_gather(srows[b], [ids, colh])
                    cold = jnp.zeros((16,), jnp.int32) + h
                    adst = plsc.load_gather(drows[b], [ids, cold])
                    e = asrc + adst
                    e = jnp.where(e >= 0.0, e, 0.2 * e)
                    p = jnp.exp(e) * ownv
                    ps.append(p)
                    plsc.store_scatter(obuf[b], [ids, colh], p)
                for j in range(CH):
                    colj = jnp.zeros((16,), jnp.int32) + j
                    v = plsc.load_gather(srows[b], [ids, colj])
                    plsc.store_scatter(obuf[b], [ids, colj], v * ps[j // OUTC])
                return cc
            lax.fori_loop(0, CHUNK // 16, group, 0)

        start_idx(0, 0)
        wait_idx(0)
        extract(0)
        start_gather(0)
        start_idx(1, 1)
        start_idx(2, 0)

        def body(i2, carry):
            for b in range(2):
                i = i2 * 2 + b
                nb = 1 - b

                @pl.when(i + 1 < nchunks)
                def _prep():
                    wait_idx(nb)

                    @pl.when(i >= 1)
                    def _():
                        wait_scat(nb)
                    extract(nb)
                    start_gather(nb)

                    @pl.when(i + 3 < nchunks)
                    def _():
                        start_idx(i + 3, nb)

                wait_gather(b)
                compute(b)
                pltpu.async_copy(obuf[b], slab.at[didx[b]], semsc[b], add=True)
            return carry
        lax.fori_loop(0, nchunks // 2, body, 0)
        wait_scat(0)
        wait_scat(1)

        plsc.subcore_barrier()
        pltpu.sync_copy(slab.at[pl.ds(r0, rpt)], acc.at[pl.ds(cN2 + r0, rpt)])

    return gat_edge


def _mm_kernel(a_ref, b_ref, o_ref):
    o_ref[...] = jnp.dot(a_ref[...], b_ref[...], preferred_element_type=jnp.float32)


def _pallas_matmul(a, b, bm=512):
    M, K = a.shape
    _, N = b.shape
    assert M % bm == 0
    return pl.pallas_call(
        _mm_kernel,
        grid=(M // bm,),
        in_specs=[
            pl.BlockSpec((bm, K), lambda i: (i, 0)),
            pl.BlockSpec((K, N), lambda i: (0, 0)),
        ],
        out_specs=pl.BlockSpec((bm, N), lambda i: (i, 0)),
        out_shape=jax.ShapeDtypeStruct((M, N), jnp.float32),
    )(a, b)


def _leaky(x):
    return jnp.where(x >= 0, x, 0.2 * x)


def _gat_sc(x, edge_index, W, a_s, a_d, b, outc):
    N = x.shape[0]
    E = edge_index.shape[1]
    ch = HEADS * outc
    chh = ch // 2
    Wt = W.reshape(W.shape[0], HEADS, outc)
    As = jnp.einsum('khc,hc->kh', Wt, a_s)
    Ad = jnp.einsum('khc,hc->kh', Wt, a_d)
    hh = _pallas_matmul(x, jnp.concatenate([W, As, Ad], axis=1))
    h = hh[:, :ch]
    als = hh[:, ch:ch + HEADS]
    ald = hh[:, ch + HEADS:ch + 2 * HEADS]
    self_ex = jnp.exp(_leaky(als + ald))

    Wrow = -(-(ch + 2 * HEADS) // 16) * 16
    hpad = jnp.zeros((N, Wrow - ch - 2 * HEADS), jnp.float32)
    ht = jnp.concatenate([hh, hpad], axis=1) if Wrow > hh.shape[1] else hh
    at = jnp.concatenate([ald, jnp.zeros((N, 12), jnp.float32)], axis=1)
    se_rep = jnp.repeat(self_ex, outc, axis=1)
    si = jnp.concatenate([h * se_rep, self_ex, self_ex, hpad], axis=1)

    epk = jnp.concatenate([
        edge_index[0].astype(jnp.int32).reshape(-1, 64),
        edge_index[1].astype(jnp.int32).reshape(-1, 64),
    ], axis=1).reshape(-1)
    acc = _make_gat_edge_sc(N, E, Wrow, ch, outc)(ht, at, si, epk)

    num = acc[:, :ch]
    den_rep = jnp.repeat(acc[:, ch:ch + HEADS], outc, axis=1) + 1e-16
    out = (num / den_rep).reshape(N, HEADS, outc).mean(axis=1) + b
    return out


def kernel(x_sg1, x_sg2, edge_index_sg1, edge_index_sg2, emb0, emb1,
           W1, a1_src, a1_dst, b1, W2, a2_src, a2_dst, b2, Wq, A):
    B = x_sg1.shape[0] // PAST
    i0a = x_sg1[:, 0].astype(jnp.int32)
    i1a = x_sg1[:, 1].astype(jnp.int32)
    x = jnp.concatenate([emb0[i0a], emb1[i1a], x_sg1[:, -3:]], axis=-1)

    x1 = _gat_sc(x, edge_index_sg1, W1, a1_src, a1_dst, b1, D1)

    i0b = x_sg2[:, 0].astype(jnp.int32)
    i1b = x_sg2[:, 1].astype(jnp.int32)
    x2_26 = jnp.concatenate([emb0[i0b], emb1[i1b], x_sg2[:, 2:4]], axis=-1)

    g1 = _pallas_matmul(x1, Wq).reshape(B, PAST, -1)
    g2 = _pallas_matmul(x2_26, Wq).reshape(B, FUTURE, -1)
    n1 = (g1 * g1).sum(-1)
    n2 = (g2 * g2).sum(-1)
    cr = jnp.einsum('bfk,bpk->bfp', g2, g1)
    logits = -(n2[:, :, None] + n1[:, None, :] - 2.0 * cr)
    A_tmp = A[:PAST, PAST:].T
    logits = jnp.where(A_tmp[None] == 0, -jnp.inf, logits)
    m = logits.max(axis=-1, keepdims=True)
    ea = jnp.exp(logits - m)
    alpha = ea / ea.sum(axis=-1, keepdims=True)
    y_past = x_sg1[:, 4].reshape(B, PAST)
    tmp = jnp.einsum('bfp,bp->bf', alpha, y_past)

    x2b = jnp.concatenate([x2_26, tmp.reshape(-1, 1)], axis=-1)
    out = _gat_sc(x2b, edge_index_sg2, W2, a2_src, a2_dst, b2, 1)
    return out.reshape(-1, FUTURE)
